# Initial kernel scaffold; baseline (speedup 1.0000x reference)
import jax
import jax.numpy as jnp
from jax.experimental import pallas as pl


def kernel(x, edge_feature, edge_index, in_proj_w, in_proj_b, out_proj_w, out_proj_b, et_w, et_b, un_w, un_b, ue_w, ue_b):
    raise NotImplementedError("write your pallas kernel here")


# speedup vs baseline: 1.0607x; 1.0607x over previous
import functools

import jax
import jax.numpy as jnp
import numpy as np
from jax.experimental import pallas as pl
from jax.experimental.pallas import tpu as pltpu
from jax.experimental.pallas import tpu_sc as plsc

_N = 10000
_E = 320000
_DN = 128
_DE = 16
_H = 2
_DH = 64
_NP = 10240


def _xpre_body(x_ref, w_ref, b_ref, o_ref):
    o_ref[...] = (
        jnp.dot(x_ref[...], w_ref[...], preferred_element_type=jnp.float32)
        + b_ref[...]
    )


def _x_pre(xp, w_t, b_row, interpret=False):
    br = 1024
    return pl.pallas_call(
        _xpre_body,
        grid=(_NP // br,),
        in_specs=[
            pl.BlockSpec((br, _DN), lambda i: (i, 0)),
            pl.BlockSpec((_DN, 416), lambda i: (0, 0)),
            pl.BlockSpec((1, 416), lambda i: (0, 0)),
        ],
        out_specs=pl.BlockSpec((br, 416), lambda i: (i, 0)),
        out_shape=jax.ShapeDtypeStruct((_NP, 416), jnp.float32),
        interpret=interpret,
    )(xp, w_t, b_row)


def _attn_body(q_ref, k_ref, v_ref, o_ref):
    q = q_ref[...]
    k = k_ref[...]
    v = v_ref[...]
    kidx = jax.lax.broadcasted_iota(jnp.int32, (q.shape[0], _NP), 1)
    outs = []
    for h in range(_H):
        qh = q[:, h * _DH:(h + 1) * _DH]
        kh = k[:, h * _DH:(h + 1) * _DH]
        vh = v[:, h * _DH:(h + 1) * _DH]
        s = jax.lax.dot_general(
            qh, kh, (((1,), (1,)), ((), ())),
            preferred_element_type=jnp.float32,
        ) * np.float32(1.0 / 8.0)
        s = jnp.where(kidx < _N, s, -1e30)
        m = jnp.max(s, axis=1, keepdims=True)
        p = jnp.exp(s - m)
        l = jnp.sum(p, axis=1, keepdims=True)
        o = jax.lax.dot_general(
            p, vh, (((1,), (0,)), ((), ())),
            preferred_element_type=jnp.float32,
        )
        outs.append(o / l)
    o_ref[...] = jnp.concatenate(outs, axis=1)


def _attention(qkv, interpret=False):
    bq = 256
    return pl.pallas_call(
        _attn_body,
        grid=(_NP // bq,),
        in_specs=[
            pl.BlockSpec((bq, _DN), lambda qi: (qi, 0)),
            pl.BlockSpec((_NP, _DN), lambda qi: (0, 1)),
            pl.BlockSpec((_NP, _DN), lambda qi: (0, 2)),
        ],
        out_specs=pl.BlockSpec((bq, _DN), lambda qi: (qi, 0)),
        out_shape=jax.ShapeDtypeStruct((_NP, _DN), jnp.float32),
        interpret=interpret,
    )(qkv, qkv, qkv)


def _epre_body(e_ref, w_ref, b_ref, o_ref):
    o = (
        jnp.dot(e_ref[...], w_ref[...], preferred_element_type=jnp.float32)
        + b_ref[...]
    )
    col = jax.lax.broadcasted_iota(jnp.int32, o.shape, 1)
    o_ref[...] = jnp.where(col % 48 >= 32, jnp.maximum(o, 0.0), o)


def _edge_pre(ef8, wbd, b_row, interpret=False):
    br = 2000
    g = _E // 8
    return pl.pallas_call(
        _epre_body,
        grid=(g // br,),
        in_specs=[
            pl.BlockSpec((br, _DN), lambda i: (i, 0)),
            pl.BlockSpec((_DN, 384), lambda i: (0, 0)),
            pl.BlockSpec((1, 384), lambda i: (0, 0)),
        ],
        out_specs=pl.BlockSpec((br, 384), lambda i: (i, 0)),
        out_shape=jax.ShapeDtypeStruct((g, 384), jnp.float32),
        interpret=interpret,
    )(ef8, wbd, b_row)


def _node_body(ao_ref, tw_ref, wo_ref, bo_ref, w1_ref, w2_ref, bn_ref, o_ref):
    xup = (
        jnp.dot(ao_ref[...], wo_ref[...], preferred_element_type=jnp.float32)
        + bo_ref[...]
    )
    o = (
        jnp.dot(xup, w1_ref[...], preferred_element_type=jnp.float32)
        + jnp.dot(tw_ref[...], w2_ref[...], preferred_element_type=jnp.float32)
        + bn_ref[...]
    )
    o_ref[...] = jnp.maximum(o, 0.0)


def _node_mlp(ao, twin, wo_t, bo_row, w1_t, w2_t, bn_row, interpret=False):
    br = 1024
    return pl.pallas_call(
        _node_body,
        grid=(_NP // br,),
        in_specs=[
            pl.BlockSpec((br, _DN), lambda i: (i, 0)),
            pl.BlockSpec((br, _DE), lambda i: (i, 0)),
            pl.BlockSpec((_DN, _DN), lambda i: (0, 0)),
            pl.BlockSpec((1, _DN), lambda i: (0, 0)),
            pl.BlockSpec((_DN, _DN), lambda i: (0, 0)),
            pl.BlockSpec((_DE, _DN), lambda i: (0, 0)),
            pl.BlockSpec((1, _DN), lambda i: (0, 0)),
        ],
        out_specs=pl.BlockSpec((br, _DN), lambda i: (i, 0)),
        out_shape=jax.ShapeDtypeStruct((_NP, _DN), jnp.float32),
        interpret=interpret,
    )(ao, twin, wo_t, bo_row, w1_t, w2_t, bn_row)


def _run(x, edge_feature, edge_index, in_proj_w, in_proj_b, out_proj_w,
         out_proj_b, et_w, et_b, un_w, un_b, ue_w, ue_b, interpret=False):
    f32 = jnp.float32
    src = edge_index[0]
    dst = edge_index[1]

    w_all = jnp.concatenate(
        [in_proj_w, ue_w[:, :_DN], ue_w[:, _DN + 2 * _DE:]], axis=0
    )
    b_all = jnp.concatenate([in_proj_b, jnp.zeros((32,), f32)])
    xp = jnp.pad(x, ((0, _NP - _N), (0, 0)))
    xpre = _x_pre(xp, w_all.T, b_all[None, :], interpret)
    qkv = xpre[:, :384]
    p1 = xpre[:, 384:400]
    p4 = xpre[:, 400:416]

    ao = _attention(qkv, interpret)

    wc = jnp.concatenate(
        [ue_w[:, _DN:_DN + _DE], ue_w[:, _DN + _DE:_DN + 2 * _DE], et_w],
        axis=0,
    )
    wbd = jnp.kron(jnp.eye(8, dtype=f32), wc.T)
    bc = jnp.tile(
        jnp.concatenate([jnp.zeros((32,), f32), et_b]), (8,)
    )
    eout = _edge_pre(
        edge_feature.reshape(_E // 8, _DN), wbd, bc[None, :], interpret
    ).reshape(_E, 48)
    ef2 = eout[:, :16]
    ef3 = eout[:, 16:32]
    tr = eout[:, 32:48]

    iota = jnp.arange(_E, dtype=jnp.int32)
    keys1 = src * _N + dst
    keys2 = dst * _N + src
    sk1, order1 = jax.lax.sort((keys1, iota), num_keys=1, is_stable=True)
    pos = jnp.clip(jnp.searchsorted(sk1, keys2), 0, _E - 1)
    found = sk1[pos] == keys2
    g = jnp.where(found, order1[pos], _E)

    ef3p = jnp.concatenate([ef3, jnp.zeros((1, _DE), f32)], axis=0)
    ue = jnp.maximum(p1[src] + ef2 + ef3p[g] + p4[dst] + ue_b[None, :], 0.0)

    subj = jnp.maximum(jax.ops.segment_max(tr, src, num_segments=_N), 0.0)
    obj = jnp.maximum(jax.ops.segment_max(tr, dst, num_segments=_N), 0.0)
    twin = jnp.pad(subj + obj, ((0, _NP - _N), (0, 0)))

    node = _node_mlp(
        ao, twin, out_proj_w.T, out_proj_b[None, :],
        un_w[:, :_DN].T, un_w[:, _DN:].T, un_b[None, :], interpret
    )
    return node[:_N], ue


def kernel(x, edge_feature, edge_index, in_proj_w, in_proj_b, out_proj_w,
           out_proj_b, et_w, et_b, un_w, un_b, ue_w, ue_b):
    return _run(x, edge_feature, edge_index, in_proj_w, in_proj_b,
                out_proj_w, out_proj_b, et_w, et_b, un_w, un_b, ue_w, ue_b)


# speedup vs baseline: 1.3082x; 1.2333x over previous
import functools

import jax
import jax.numpy as jnp
import numpy as np
from jax.experimental import pallas as pl
from jax.experimental.pallas import tpu as pltpu
from jax.experimental.pallas import tpu_sc as plsc

_N = 10000
_E = 320000
_DN = 128
_DE = 16
_H = 2
_DH = 64
_NP = 10240


def _xpre_body(x_ref, w_ref, b_ref, o_ref):
    o_ref[...] = (
        jnp.dot(x_ref[...], w_ref[...], preferred_element_type=jnp.float32)
        + b_ref[...]
    )


def _x_pre(xp, w_t, b_row, interpret=False):
    br = 1024
    return pl.pallas_call(
        _xpre_body,
        grid=(_NP // br,),
        in_specs=[
            pl.BlockSpec((br, _DN), lambda i: (i, 0)),
            pl.BlockSpec((_DN, 416), lambda i: (0, 0)),
            pl.BlockSpec((1, 416), lambda i: (0, 0)),
        ],
        out_specs=pl.BlockSpec((br, 416), lambda i: (i, 0)),
        out_shape=jax.ShapeDtypeStruct((_NP, 416), jnp.float32),
        interpret=interpret,
    )(xp, w_t, b_row)


def _attn_body(q_ref, k_ref, v_ref, o_ref):
    q = q_ref[...]
    k = k_ref[...]
    v = v_ref[...]
    kidx = jax.lax.broadcasted_iota(jnp.int32, (q.shape[0], _NP), 1)
    outs = []
    for h in range(_H):
        qh = q[:, h * _DH:(h + 1) * _DH]
        kh = k[:, h * _DH:(h + 1) * _DH]
        vh = v[:, h * _DH:(h + 1) * _DH]
        s = jax.lax.dot_general(
            qh, kh, (((1,), (1,)), ((), ())),
            preferred_element_type=jnp.float32,
        ) * np.float32(1.0 / 8.0)
        s = jnp.where(kidx < _N, s, -1e30)
        m = jnp.max(s, axis=1, keepdims=True)
        p = jnp.exp(s - m)
        l = jnp.sum(p, axis=1, keepdims=True)
        o = jax.lax.dot_general(
            p, vh, (((1,), (0,)), ((), ())),
            preferred_element_type=jnp.float32,
        )
        outs.append(o / l)
    o_ref[...] = jnp.concatenate(outs, axis=1)


def _attention(qkv, interpret=False):
    bq = 256
    return pl.pallas_call(
        _attn_body,
        grid=(_NP // bq,),
        in_specs=[
            pl.BlockSpec((bq, _DN), lambda qi: (qi, 0)),
            pl.BlockSpec((_NP, _DN), lambda qi: (0, 1)),
            pl.BlockSpec((_NP, _DN), lambda qi: (0, 2)),
        ],
        out_specs=pl.BlockSpec((bq, _DN), lambda qi: (qi, 0)),
        out_shape=jax.ShapeDtypeStruct((_NP, _DN), jnp.float32),
        interpret=interpret,
    )(qkv, qkv, qkv)


def _epre_body(e_ref, w_ref, b_ref, o_ref):
    o = (
        jnp.dot(e_ref[...], w_ref[...], preferred_element_type=jnp.float32)
        + b_ref[...]
    )
    col = jax.lax.broadcasted_iota(jnp.int32, o.shape, 1)
    o_ref[...] = jnp.where(col % 48 >= 32, jnp.maximum(o, 0.0), o)


def _edge_pre(ef8, wbd, b_row, interpret=False):
    br = 2000
    g = _E // 8
    return pl.pallas_call(
        _epre_body,
        grid=(g // br,),
        in_specs=[
            pl.BlockSpec((br, _DN), lambda i: (i, 0)),
            pl.BlockSpec((_DN, 384), lambda i: (0, 0)),
            pl.BlockSpec((1, 384), lambda i: (0, 0)),
        ],
        out_specs=pl.BlockSpec((br, 384), lambda i: (i, 0)),
        out_shape=jax.ShapeDtypeStruct((g, 384), jnp.float32),
        interpret=interpret,
    )(ef8, wbd, b_row)


def _node_body(ao_ref, tw_ref, wo_ref, bo_ref, w1_ref, w2_ref, bn_ref, o_ref):
    xup = (
        jnp.dot(ao_ref[...], wo_ref[...], preferred_element_type=jnp.float32)
        + bo_ref[...]
    )
    o = (
        jnp.dot(xup, w1_ref[...], preferred_element_type=jnp.float32)
        + jnp.dot(tw_ref[...], w2_ref[...], preferred_element_type=jnp.float32)
        + bn_ref[...]
    )
    o_ref[...] = jnp.maximum(o, 0.0)


def _node_mlp(ao, twin, wo_t, bo_row, w1_t, w2_t, bn_row, interpret=False):
    br = 1024
    return pl.pallas_call(
        _node_body,
        grid=(_NP // br,),
        in_specs=[
            pl.BlockSpec((br, _DN), lambda i: (i, 0)),
            pl.BlockSpec((br, _DE), lambda i: (i, 0)),
            pl.BlockSpec((_DN, _DN), lambda i: (0, 0)),
            pl.BlockSpec((1, _DN), lambda i: (0, 0)),
            pl.BlockSpec((_DN, _DN), lambda i: (0, 0)),
            pl.BlockSpec((_DE, _DN), lambda i: (0, 0)),
            pl.BlockSpec((1, _DN), lambda i: (0, 0)),
        ],
        out_specs=pl.BlockSpec((br, _DN), lambda i: (i, 0)),
        out_shape=jax.ShapeDtypeStruct((_NP, _DN), jnp.float32),
        interpret=interpret,
    )(ao, twin, wo_t, bo_row, w1_t, w2_t, bn_row)


_C = 512
_CA = 400
_NW = 32
_NPT = _NP // _NW
_EP = _E // _NW


def _segmax_body(tr_h, o1_h, k1_h, o2_h, k2_h, bnd_h, out_h,
                 acc_s, acc_o, idx_v, rows_v, key_v, bnd_v, obuf, sem):
    wid = jax.lax.axis_index("s") * 2 + jax.lax.axis_index("c")
    nbase = wid * _NPT
    pltpu.sync_copy(bnd_h, bnd_v)

    def one_pass(o_h, k_h, lo, hi, acc):
        def zero(i, _):
            acc[i] = jnp.zeros((16,), jnp.float32)
            return 0
        jax.lax.fori_loop(0, _NPT, zero, 0)
        la = (lo // 8) * 8
        nch = (hi - la + _C - 1) // _C

        def chunk(c, _):
            e0 = la + c * _C
            pltpu.sync_copy(o_h.at[pl.ds(e0, _C)], idx_v)
            pltpu.async_copy(tr_h.at[idx_v], rows_v, sem).wait()
            pltpu.sync_copy(k_h.at[pl.ds(e0, _C)], key_v.at[pl.ds(0, _C)])
            s_i = jnp.maximum(lo - e0, 0)
            e_i = jnp.minimum(hi - e0, _C)

            def upd(i, _):
                kv = key_v[pl.ds(i, 16)]
                n = kv[0] // _N - nbase
                acc[n] = jnp.maximum(acc[n], rows_v[i])
                return 0
            jax.lax.fori_loop(s_i, e_i, upd, 0)
            return 0
        jax.lax.fori_loop(0, nch, chunk, 0)

    bv1 = bnd_v[pl.ds(wid, 16)]
    one_pass(o1_h, k1_h, bv1[0], bv1[1], acc_s)
    bv2 = bnd_v[pl.ds(36 + wid, 16)]
    one_pass(o2_h, k2_h, bv2[0], bv2[1], acc_o)

    def comb(i, _):
        obuf[i] = acc_s[i] + acc_o[i]
        return 0
    jax.lax.fori_loop(0, _NPT, comb, 0)
    pltpu.sync_copy(obuf, out_h.at[pl.ds(nbase, _NPT)])


def _segmax(tr, o1p, k1p, o2p, k2p, bnd_all, interpret=False):
    mesh = plsc.VectorSubcoreMesh(core_axis_name="c", subcore_axis_name="s", num_cores=2, num_subcores=16)
    f = pl.kernel(
        _segmax_body,
        out_type=jax.ShapeDtypeStruct((_NP, _DE), jnp.float32),
        mesh=mesh,
        scratch_types=[
            pltpu.VMEM((_NPT, _DE), jnp.float32),
            pltpu.VMEM((_NPT, _DE), jnp.float32),
            pltpu.VMEM((_C,), jnp.int32),
            pltpu.VMEM((_C, _DE), jnp.float32),
            pltpu.VMEM((_C + 16,), jnp.int32),
            pltpu.VMEM((88,), jnp.int32),
            pltpu.VMEM((_NPT, _DE), jnp.float32),
            pltpu.SemaphoreType.DMA,
        ],
        compiler_params=pltpu.CompilerParams(use_tc_tiling_on_sc=False),
        interpret=interpret,
    )
    return f(tr, o1p, k1p, o2p, k2p, bnd_all)


_S = _E // 16
_CS = 400


def _revsearch_body(q_h, sub_h, skw_h, ow_h, out_h,
                    sub_v, q_v, rix_v, wink_v, wino_v, g_v, sem, sem2):
    wid = jax.lax.axis_index("s") * 2 + jax.lax.axis_index("c")
    ebase = wid * _EP
    pltpu.sync_copy(sub_h, sub_v)
    pltpu.sync_copy(q_h.at[pl.ds(ebase, _EP)], q_v)
    lanes = jax.lax.iota(jnp.int32, 16)

    def chunk(c, _):
        c0 = c * _CS

        def search(v, _):
            gidx = c0 + v * 16 + lanes
            q16 = q_v[pl.ds(c0 + v * 16, 16)]
            lo = jnp.zeros((16,), jnp.int32)
            hi = jnp.full((16,), _S, jnp.int32)

            def step(t, carry):
                lo, hi = carry
                mid = (lo + hi) >> 1
                val = plsc.load_gather(sub_v, [mid])
                pred = val < q16
                return (jnp.where(pred, mid + 1, lo),
                        jnp.where(pred, hi, mid))
            lo, hi = jax.lax.fori_loop(0, 15, step, (lo, hi))
            r = jnp.minimum(jnp.maximum(lo - 1, 0), _S - 2)
            loc2 = (v * 16 + lanes) * 2
            plsc.store_scatter(rix_v, [loc2], r)
            plsc.store_scatter(rix_v, [loc2 + 1], r + 1)
            return 0
        jax.lax.fori_loop(0, _CS // 16, search, 0)
        cpk = pltpu.async_copy(skw_h.at[rix_v], wink_v, sem)
        cpo = pltpu.async_copy(ow_h.at[rix_v], wino_v, sem2)
        cpk.wait()
        cpo.wait()

        def refine(v, _):
            q16 = q_v[pl.ds(c0 + v * 16, 16)]
            row0 = (v * 16 + lanes) * 2
            lo2 = jnp.zeros((16,), jnp.int32)
            hi2 = jnp.full((16,), 32, jnp.int32)

            def step2(t, carry):
                lo2, hi2 = carry
                mid = (lo2 + hi2) >> 1
                val = plsc.load_gather(
                    wink_v, [row0 + (mid >> 4), mid & 15]
                )
                pred = val < q16
                return (jnp.where(pred, mid + 1, lo2),
                        jnp.where(pred, hi2, mid))
            lo2, hi2 = jax.lax.fori_loop(0, 5, step2, (lo2, hi2))
            relc = jnp.minimum(lo2, 31)
            keyat = plsc.load_gather(
                wink_v, [row0 + (relc >> 4), relc & 15]
            )
            gval = plsc.load_gather(
                wino_v, [row0 + (relc >> 4), relc & 15]
            )
            found = jnp.logical_and(lo2 < 32, keyat == q16)
            g_v[pl.ds(v * 16, 16)] = jnp.where(found, gval, _E)
            return 0
        jax.lax.fori_loop(0, _CS // 16, refine, 0)
        pltpu.sync_copy(g_v, out_h.at[pl.ds(ebase + c0, _CS)])
        return 0
    jax.lax.fori_loop(0, _EP // _CS, chunk, 0)


def _revsearch(keys2, sub, skw, ow, interpret=False):
    mesh = plsc.VectorSubcoreMesh(core_axis_name="c", subcore_axis_name="s",
                                  num_cores=2, num_subcores=16)
    f = pl.kernel(
        _revsearch_body,
        out_type=jax.ShapeDtypeStruct((_E,), jnp.int32),
        mesh=mesh,
        scratch_types=[
            pltpu.VMEM((_S,), jnp.int32),
            pltpu.VMEM((_EP,), jnp.int32),
            pltpu.VMEM((2 * _CS,), jnp.int32),
            pltpu.VMEM((2 * _CS, 16), jnp.int32),
            pltpu.VMEM((2 * _CS, 16), jnp.int32),
            pltpu.VMEM((_CS,), jnp.int32),
            pltpu.SemaphoreType.DMA,
            pltpu.SemaphoreType.DMA,
        ],
        compiler_params=pltpu.CompilerParams(use_tc_tiling_on_sc=False),
        interpret=interpret,
    )
    return f(keys2, sub, skw, ow)


def _assemble_body(src_h, dst_h, g_h, p1_h, p4_h, ef2_h, ef3_h, b_h, out_h,
                   i1_v, i4_v, i3_v, a1_v, a4_v, a3_v, a2_v, o_v, b_v,
                   s1, s2, s3, s4):
    wid = jax.lax.axis_index("s") * 2 + jax.lax.axis_index("c")
    ebase = wid * _EP
    pltpu.sync_copy(b_h, b_v)
    bias = b_v[...]

    def chunk(c, _):
        e0 = ebase + c * _CA
        pltpu.sync_copy(src_h.at[pl.ds(e0, _CA)], i1_v)
        pltpu.sync_copy(dst_h.at[pl.ds(e0, _CA)], i4_v)
        pltpu.sync_copy(g_h.at[pl.ds(e0, _CA)], i3_v)
        cp1 = pltpu.async_copy(p1_h.at[i1_v], a1_v, s1)
        cp4 = pltpu.async_copy(p4_h.at[i4_v], a4_v, s2)
        cp3 = pltpu.async_copy(ef3_h.at[i3_v], a3_v, s3)
        cp2 = pltpu.async_copy(ef2_h.at[pl.ds(e0, _CA)], a2_v, s4)
        cp1.wait()
        cp4.wait()
        cp3.wait()
        cp2.wait()

        def rowf(i, _):
            o_v[i] = jnp.maximum(
                a1_v[i] + a2_v[i] + a3_v[i] + a4_v[i] + bias, 0.0
            )
            return 0
        jax.lax.fori_loop(0, _CA, rowf, 0)
        pltpu.sync_copy(o_v, out_h.at[pl.ds(e0, _CA)])
        return 0
    jax.lax.fori_loop(0, _EP // _CA, chunk, 0)


def _assemble(srcv, dstv, g, p1, p4, ef2, ef3p, bias, interpret=False):
    mesh = plsc.VectorSubcoreMesh(core_axis_name="c", subcore_axis_name="s", num_cores=2, num_subcores=16)
    f = pl.kernel(
        _assemble_body,
        out_type=jax.ShapeDtypeStruct((_E, _DE), jnp.float32),
        mesh=mesh,
        scratch_types=[
            pltpu.VMEM((_CA,), jnp.int32),
            pltpu.VMEM((_CA,), jnp.int32),
            pltpu.VMEM((_CA,), jnp.int32),
            pltpu.VMEM((_CA, _DE), jnp.float32),
            pltpu.VMEM((_CA, _DE), jnp.float32),
            pltpu.VMEM((_CA, _DE), jnp.float32),
            pltpu.VMEM((_CA, _DE), jnp.float32),
            pltpu.VMEM((_CA, _DE), jnp.float32),
            pltpu.VMEM((_DE,), jnp.float32),
            pltpu.SemaphoreType.DMA,
            pltpu.SemaphoreType.DMA,
            pltpu.SemaphoreType.DMA,
            pltpu.SemaphoreType.DMA,
        ],
        compiler_params=pltpu.CompilerParams(use_tc_tiling_on_sc=False),
        interpret=interpret,
    )
    return f(srcv, dstv, g, p1, p4, ef2, ef3p, bias)


def _run(x, edge_feature, edge_index, in_proj_w, in_proj_b, out_proj_w,
         out_proj_b, et_w, et_b, un_w, un_b, ue_w, ue_b, interpret=False):
    f32 = jnp.float32
    src = edge_index[0]
    dst = edge_index[1]

    w_all = jnp.concatenate(
        [in_proj_w, ue_w[:, :_DN], ue_w[:, _DN + 2 * _DE:]], axis=0
    )
    b_all = jnp.concatenate([in_proj_b, jnp.zeros((32,), f32)])
    xp = jnp.pad(x, ((0, _NP - _N), (0, 0)))
    xpre = _x_pre(xp, w_all.T, b_all[None, :], interpret)
    qkv = xpre[:, :384]
    p1 = xpre[:, 384:400]
    p4 = xpre[:, 400:416]

    ao = _attention(qkv, interpret)

    wc = jnp.concatenate(
        [ue_w[:, _DN:_DN + _DE], ue_w[:, _DN + _DE:_DN + 2 * _DE], et_w],
        axis=0,
    )
    wbd = jnp.kron(jnp.eye(8, dtype=f32), wc.T)
    bc = jnp.tile(
        jnp.concatenate([jnp.zeros((32,), f32), et_b]), (8,)
    )
    eout = _edge_pre(
        edge_feature.reshape(_E // 8, _DN), wbd, bc[None, :], interpret
    ).reshape(_E, 48)
    ef2 = eout[:, :16]
    ef3 = eout[:, 16:32]
    tr = eout[:, 32:48]

    iota = jnp.arange(_E, dtype=jnp.int32)
    keys1 = src * _N + dst
    keys2 = dst * _N + src
    sk1, order1 = jax.lax.sort((keys1, iota), num_keys=1, is_stable=True)
    sk2, order2 = jax.lax.sort((keys2, iota), num_keys=1, is_stable=True)
    pos = jnp.clip(jnp.searchsorted(sk1, keys2), 0, _E - 1)
    found = sk1[pos] == keys2
    g = jnp.where(found, order1[pos], _E).astype(jnp.int32)

    ef3p = jnp.concatenate([ef3, jnp.zeros((8, _DE), f32)], axis=0)
    ue = _assemble(src, dst, g, p1, p4, ef2, ef3p, ue_b, interpret)

    nodespan = jnp.arange(33, dtype=jnp.int32) * (_NPT * _N)
    bnd1 = jnp.searchsorted(sk1, nodespan).astype(jnp.int32)
    bnd2 = jnp.searchsorted(sk2, nodespan).astype(jnp.int32)
    zpad3 = jnp.zeros((3,), jnp.int32)
    bnd_all = jnp.concatenate(
        [bnd1, zpad3, bnd2, zpad3, jnp.zeros((16,), jnp.int32)]
    )
    twin = _segmax(
        tr,
        jnp.pad(order1, (0, _C)), jnp.pad(sk1, (0, _C)),
        jnp.pad(order2, (0, _C)), jnp.pad(sk2, (0, _C)),
        bnd_all, interpret,
    )

    node = _node_mlp(
        ao, twin, out_proj_w.T, out_proj_b[None, :],
        un_w[:, :_DN].T, un_w[:, _DN:].T, un_b[None, :], interpret
    )
    return node[:_N], ue


def kernel(x, edge_feature, edge_index, in_proj_w, in_proj_b, out_proj_w,
           out_proj_b, et_w, et_b, un_w, un_b, ue_w, ue_b):
    return _run(x, edge_feature, edge_index, in_proj_w, in_proj_b,
                out_proj_w, out_proj_b, et_w, et_b, un_w, un_b, ue_w, ue_b)
